# baseline (speedup 1.0000x reference)
import functools

import jax
import jax.numpy as jnp
import numpy as np
from jax import lax
from jax.experimental import pallas as pl
from jax.experimental.pallas import tpu as pltpu
from jax.experimental.pallas import tpu_sc as plsc

NMC = 8
L = 32
GRID = NMC * L * L
ROWS = 2 * GRID
LANES = 16
BG = 128
WIN = 4096
CH = 256

_EDGES = tuple(np.float32(10.0 ** (j / 4.0)) for j in range(1, NMC))


def _make_sc_call(n_batch, n):
    assert n % LANES == 0
    groups = n // LANES
    n_chunks = WIN // CH

    mesh = plsc.VectorSubcoreMesh(core_axis_name="c", subcore_axis_name="s")

    @functools.partial(
        pl.kernel,
        mesh=mesh,
        compiler_params=pltpu.CompilerParams(needs_layout_passes=False),
        out_type=jax.ShapeDtypeStruct((ROWS, n_batch), jnp.float32),
        scratch_types=[
            pltpu.VMEM((CH, BG), jnp.float32),
            pltpu.VMEM((CH, BG), jnp.float32),
            pltpu.VMEM((BG, n), jnp.float32),
            pltpu.VMEM((BG, n), jnp.float32),
            pltpu.VMEM((BG, n), jnp.float32),
            pltpu.VMEM((16 * 768,), jnp.int32),
            pltpu.VMEM((4 * LANES,), jnp.int32),
            pltpu.SemaphoreType.DMA,
            pltpu.SemaphoreType.DMA,
            pltpu.SemaphoreType.DMA,
        ],
    )
    def sc_kernel(xs_hbm, ys_hbm, ms_hbm, out_hbm, buf0, buf1, xv, yv, mv,
                  ptsb, cnt, sem2, sem0, sem1):
        wid = lax.axis_index("s") * 2 + lax.axis_index("c")
        g = wid // 4
        p = wid % 4

        ones_f = jnp.full((LANES,), 1.0, jnp.float32)
        zeros_f = jnp.zeros((LANES,), jnp.float32)
        one_i = jnp.ones((LANES,), jnp.int32)

        is_ones_half = jnp.broadcast_to((wid & 2) == 0, (LANES,))
        bg_vec = jnp.where(is_ones_half, ones_f, zeros_f)
        val_vec = jnp.where(is_ones_half, zeros_f, ones_f)

        in_x = pltpu.async_copy(xs_hbm.at[pl.ds(g * BG, BG)], xv, sem0)
        in_y = pltpu.async_copy(ys_hbm.at[pl.ds(g * BG, BG)], yv, sem1)
        in_m = pltpu.async_copy(ms_hbm.at[pl.ds(g * BG, BG)], mv, sem2)

        neg = jnp.full((LANES,), -1, jnp.int32)

        def fill(r, _):
            for cb in range(BG // LANES):
                buf0[r, pl.ds(cb * LANES, LANES)] = bg_vec
                buf1[r, pl.ds(cb * LANES, LANES)] = bg_vec
            for cb in range(3):
                ptsb[pl.ds(r * 3 * LANES + cb * LANES, LANES)] = neg
            return _

        with jax.named_scope("ph_fill"):
            lax.fori_loop(0, CH, fill, None)
        zi = jnp.zeros((LANES,), jnp.int32)
        for q in range(4):
            cnt[pl.ds(q * LANES, LANES)] = zi
        in_x.wait()
        in_y.wait()
        in_m.wait()

        rel_off = (wid & 1) * WIN
        CAP = 768
        SUB = CAP // 4

        neg_i = jnp.full((LANES,), -1, jnp.int32)

        def points_pack(j, _):
            for k in range(groups):
                x = xv[j, pl.ds(k * LANES, LANES)]
                y = yv[j, pl.ds(k * LANES, LANES)]
                m = mv[j, pl.ds(k * LANES, LANES)]
                xi = (x * np.float32(L)).astype(jnp.int32)
                yi = (y * np.float32(L)).astype(jnp.int32)
                mi = jnp.zeros((LANES,), jnp.int32)
                for e in _EDGES:
                    mi = mi + jnp.where(m >= e, one_i, 0)
                rel = mi * (L * L) + yi * L + xi - rel_off
                valid = (rel >= 0) & (rel < WIN)
                pk = jnp.where(valid, rel * BG + j, neg_i)
                xv[j, pl.ds(k * LANES, LANES)] = plsc.bitcast(pk, jnp.float32)
            return _

        def bucketize(j, _):
            for k in range(groups):
                pk = plsc.bitcast(xv[j, pl.ds(k * LANES, LANES)], jnp.int32)
                valid = pk >= 0
                cid = (pk >> 15) & (n_chunks - 1)
                rank, last = plsc.scan_count(cid, mask=valid)
                base = plsc.load_gather(cnt, [k * LANES + cid], mask=valid)
                slot = cid * CAP + k * SUB + base + rank - 1
                plsc.store_scatter(ptsb, [slot], pk, mask=valid)
                plsc.store_scatter(cnt, [k * LANES + cid], base + rank,
                                   mask=valid & last)
            return _

        with jax.named_scope("ph_pack"):
            lax.fori_loop(0, BG, points_pack, None)
        with jax.named_scope("ph_bucketize"):
            lax.fori_loop(0, BG, bucketize, None)
        cnts = [cnt[pl.ds(q * LANES, LANES)] for q in range(groups)]

        def bucket_pass(c, buf, vec):
            for q in range(groups):
                trip = (cnts[q][c] + LANES - 1) // LANES

                def body(i, _):
                    pk = ptsb[pl.ds(c * CAP + q * SUB + i * LANES, LANES)]
                    row = (pk >> 7) & (CH - 1)
                    col = pk & (BG - 1)
                    plsc.store_scatter(buf, [row, col], vec, mask=(pk >= 0))
                    return _

                lax.fori_loop(0, trip, body, None)

        bufs = (buf0, buf1)
        sems = (sem0, sem1)
        copies = [None, None]
        for c in range(n_chunks):
          with jax.named_scope(f"ph_chunk{c}"):
              k2 = c % 2
              buf = bufs[k2]
              if copies[k2] is not None:
                  copies[k2].wait()
                  bucket_pass(c - 2, buf, bg_vec)
              bucket_pass(c, buf, val_vec)
              copies[k2] = pltpu.async_copy(
                  buf, out_hbm.at[pl.ds(p * WIN + c * CH, CH), pl.ds(g * BG, BG)],
                  sems[k2])
        for k2 in range(2):
            copies[k2].wait()

    return sc_kernel


@jax.jit
def kernel(coord_v):
    n_batch = coord_v.shape[0]
    n = coord_v.shape[1] // 3
    c = coord_v.reshape(n_batch, n, 3)
    xs = c[:, :, 0]
    ys = c[:, :, 1]
    ms = c[:, :, 2]
    out = _make_sc_call(n_batch, n)(xs, ys, ms)
    return out.reshape(2 * NMC, L, L, n_batch).transpose(3, 0, 1, 2)


# speedup vs baseline: 1.0550x; 1.0550x over previous
import functools

import jax
import jax.numpy as jnp
import numpy as np
from jax import lax
from jax.experimental import pallas as pl
from jax.experimental.pallas import tpu as pltpu
from jax.experimental.pallas import tpu_sc as plsc

NMC = 8
L = 32
GRID = NMC * L * L
ROWS = 2 * GRID
LANES = 16
BG = 128
WIN = 4096
CH = 256

_EDGES = tuple(np.float32(10.0 ** (j / 4.0)) for j in range(1, NMC))


def _make_sc_call(n_batch, n):
    assert n % LANES == 0
    groups = n // LANES
    n_chunks = WIN // CH

    mesh = plsc.VectorSubcoreMesh(core_axis_name="c", subcore_axis_name="s")

    @functools.partial(
        pl.kernel,
        mesh=mesh,
        compiler_params=pltpu.CompilerParams(needs_layout_passes=False),
        out_type=jax.ShapeDtypeStruct((ROWS, n_batch), jnp.float32),
        scratch_types=[
            pltpu.VMEM((CH, BG), jnp.float32),
            pltpu.VMEM((CH, BG), jnp.float32),
            pltpu.VMEM((BG, n), jnp.float32),
            pltpu.VMEM((BG, n), jnp.float32),
            pltpu.VMEM((BG, n), jnp.float32),
            pltpu.VMEM((16 * 768,), jnp.int32),
            pltpu.VMEM((4 * LANES,), jnp.int32),
            pltpu.SemaphoreType.DMA,
            pltpu.SemaphoreType.DMA,
            pltpu.SemaphoreType.DMA,
        ],
    )
    def sc_kernel(xs_hbm, ys_hbm, ms_hbm, out_hbm, buf0, buf1, xv, yv, mv,
                  ptsb, cnt, sem2, sem0, sem1):
        wid = lax.axis_index("s") * 2 + lax.axis_index("c")
        g = wid // 4
        p = wid % 4

        ones_f = jnp.full((LANES,), 1.0, jnp.float32)
        zeros_f = jnp.zeros((LANES,), jnp.float32)
        one_i = jnp.ones((LANES,), jnp.int32)

        is_ones_half = jnp.broadcast_to((wid & 2) == 0, (LANES,))
        bg_vec = jnp.where(is_ones_half, ones_f, zeros_f)
        val_vec = jnp.where(is_ones_half, zeros_f, ones_f)

        in_x = pltpu.async_copy(xs_hbm.at[pl.ds(g * BG, BG)], xv, sem0)
        in_y = pltpu.async_copy(ys_hbm.at[pl.ds(g * BG, BG)], yv, sem1)
        in_m = pltpu.async_copy(ms_hbm.at[pl.ds(g * BG, BG)], mv, sem2)

        neg = jnp.full((LANES,), -1, jnp.int32)

        def fill(r, _):
            for cb in range(BG // LANES):
                buf0[r, pl.ds(cb * LANES, LANES)] = bg_vec
                buf1[r, pl.ds(cb * LANES, LANES)] = bg_vec
            for cb in range(3):
                ptsb[pl.ds(r * 3 * LANES + cb * LANES, LANES)] = neg
            return _

        lax.fori_loop(0, CH, fill, None)
        zi = jnp.zeros((LANES,), jnp.int32)
        for q in range(4):
            cnt[pl.ds(q * LANES, LANES)] = zi
        in_x.wait()
        in_y.wait()
        in_m.wait()

        rel_off = (wid & 1) * WIN
        CAP = 768
        SUB = CAP // 4

        def precompute(j, _):
            for k in range(groups):
                x = xv[j, pl.ds(k * LANES, LANES)]
                y = yv[j, pl.ds(k * LANES, LANES)]
                m = mv[j, pl.ds(k * LANES, LANES)]
                xi = (x * np.float32(L)).astype(jnp.int32)
                yi = (y * np.float32(L)).astype(jnp.int32)
                mi = jnp.zeros((LANES,), jnp.int32)
                for e in _EDGES:
                    mi = mi + jnp.where(m >= e, one_i, 0)
                rel = mi * (L * L) + yi * L + xi - rel_off
                valid = (rel >= 0) & (rel < WIN)
                pk = rel * BG + j
                cid = (pk >> 15) & (n_chunks - 1)
                rank, last = plsc.scan_count(cid, mask=valid)
                base = plsc.load_gather(cnt, [k * LANES + cid], mask=valid)
                slot = cid * CAP + k * SUB + base + rank - 1
                plsc.store_scatter(ptsb, [slot], pk, mask=valid)
                plsc.store_scatter(cnt, [k * LANES + cid], base + rank,
                                   mask=valid & last)
            return _

        lax.fori_loop(0, BG, precompute, None)
        cnts = [cnt[pl.ds(q * LANES, LANES)] for q in range(groups)]

        def bucket_pass(c, buf, vec):
            for q in range(groups):
                trip = (cnts[q][c] + LANES - 1) // LANES

                def body(i, _):
                    pk = ptsb[pl.ds(c * CAP + q * SUB + i * LANES, LANES)]
                    row = (pk >> 7) & (CH - 1)
                    col = pk & (BG - 1)
                    plsc.store_scatter(buf, [row, col], vec, mask=(pk >= 0))
                    return _

                lax.fori_loop(0, trip, body, None)

        bufs = (buf0, buf1)
        sems = (sem0, sem1)
        copies = [None, None]
        for c in range(n_chunks):
            k2 = c % 2
            buf = bufs[k2]
            if copies[k2] is not None:
                copies[k2].wait()
                bucket_pass(c - 2, buf, bg_vec)
            bucket_pass(c, buf, val_vec)
            copies[k2] = pltpu.async_copy(
                buf, out_hbm.at[pl.ds(p * WIN + c * CH, CH), pl.ds(g * BG, BG)],
                sems[k2])
        for k2 in range(2):
            copies[k2].wait()

    return sc_kernel


@jax.jit
def kernel(coord_v):
    n_batch = coord_v.shape[0]
    n = coord_v.shape[1] // 3
    c = coord_v.reshape(n_batch, n, 3)
    xs = c[:, :, 0]
    ys = c[:, :, 1]
    ms = c[:, :, 2]
    out = _make_sc_call(n_batch, n)(xs, ys, ms)
    return out.reshape(2 * NMC, L, L, n_batch).transpose(3, 0, 1, 2)
